# Initial kernel scaffold; baseline (speedup 1.0000x reference)
import jax
import jax.numpy as jnp
from jax.experimental import pallas as pl


def kernel(points):
    raise NotImplementedError("write your pallas kernel here")


# speedup vs baseline: 1.1838x; 1.1838x over previous
import functools

import jax
import jax.numpy as jnp
from jax.experimental import pallas as pl
from jax.experimental.pallas import tpu as pltpu

K = 16


def _fps_kernel(pts_t_ref, out_ref):
    B, _, N = pts_t_ref.shape
    M = out_ref.shape[1]
    X = pts_t_ref[:, 0, :]
    Y = pts_t_ref[:, 1, :]
    Z = pts_t_ref[:, 2, :]
    iota = jax.lax.broadcasted_iota(jnp.int32, (B, N), 1)

    def row(cx, cy, cz):
        return jnp.concatenate(
            [cx.reshape(B, 1, 1), cy.reshape(B, 1, 1), cz.reshape(B, 1, 1)],
            axis=2)

    lx, ly, lz = X[:, 0:1], Y[:, 0:1], Z[:, 0:1]
    out_ref[:, 0:1, :] = row(lx, ly, lz)
    mind0 = jnp.full((B, N), 1e10, dtype=jnp.float32)

    def step(j, carry):
        mind, lx, ly, lz = carry
        dx = X - lx
        dy = Y - ly
        dz = Z - lz
        d = (dx * dx + dz * dz) + dy * dy
        mind = jnp.minimum(mind, d)
        m = jnp.max(mind, axis=1, keepdims=True)
        am = jnp.min(jnp.where(mind == m, iota, N), axis=1, keepdims=True)
        oh = iota == am
        nlx = jnp.sum(jnp.where(oh, X, 0.0), axis=1, keepdims=True)
        nly = jnp.sum(jnp.where(oh, Y, 0.0), axis=1, keepdims=True)
        nlz = jnp.sum(jnp.where(oh, Z, 0.0), axis=1, keepdims=True)
        out_ref[:, pl.ds(j, 1), :] = row(nlx, nly, nlz)
        return (mind, nlx, nly, nlz)

    jax.lax.fori_loop(1, M, step, (mind0, lx, ly, lz), unroll=False)


def _fps(pts_t, m):
    B = pts_t.shape[0]
    return pl.pallas_call(
        _fps_kernel,
        out_shape=jax.ShapeDtypeStruct((B, m, 3), jnp.float32),
    )(pts_t)


def _sqnorm(x, y, z, order):
    if order == "fwd":
        return (x * x + y * y) + z * z
    return (x * x + z * z) + y * y


def _knn_kernel(q_ref, st_ref, o_ref, *, q_order, s_order):
    R = q_ref.shape[1]
    S = st_ref.shape[2]
    qx, qy, qz = q_ref[0, :, 0:1], q_ref[0, :, 1:2], q_ref[0, :, 2:3]
    sx, sy, sz = st_ref[0, 0:1, :], st_ref[0, 1:2, :], st_ref[0, 2:3, :]
    q2 = _sqnorm(qx, qy, qz, q_order)
    s2 = _sqnorm(sx, sy, sz, s_order)
    qs = jax.lax.dot_general(
        q_ref[0], st_ref[0], (((1,), (0,)), ((), ())),
        precision=jax.lax.Precision.DEFAULT,
        preferred_element_type=jnp.float32)
    d = (q2 + s2) - 2.0 * qs
    iota = jax.lax.broadcasted_iota(jnp.int32, (R, S), 1)
    big = jnp.float32(jnp.inf)
    for j in range(K):
        m = jnp.min(d, axis=1, keepdims=True)
        am = jnp.min(jnp.where(d == m, iota, S), axis=1, keepdims=True)
        o_ref[0, :, j:j + 1] = am
        if j + 1 < K:
            d = jnp.where(iota == am, big, d)


def _knn(q_rows, st, q_order, s_order):
    B, Q, _ = q_rows.shape
    S = st.shape[2]
    R = 8
    return pl.pallas_call(
        functools.partial(_knn_kernel, q_order=q_order, s_order=s_order),
        grid=(B, Q // R),
        in_specs=[
            pl.BlockSpec((1, R, 3), lambda b, i: (b, i, 0)),
            pl.BlockSpec((1, 3, S), lambda b, i: (b, 0, 0)),
        ],
        out_specs=pl.BlockSpec((1, R, K), lambda b, i: (b, i, 0)),
        out_shape=jax.ShapeDtypeStruct((B, Q, K), jnp.int32),
    )(q_rows, st)


def kernel(points):
    pts = points[..., :3]
    B, n0, _ = pts.shape
    n1, n2 = n0 // 4, n0 // 16
    pts_t = jnp.swapaxes(pts, 1, 2)
    p1 = _fps(pts_t, n1)
    p1_t = jnp.swapaxes(p1, 1, 2)
    p2 = _fps(p1_t, n2)
    p2_t = jnp.swapaxes(p2, 1, 2)
    nb0 = _knn(pts, pts_t, "fwd", "fwd")
    nb1 = _knn(p1, p1_t, "mid", "mid")
    nb2 = _knn(p2, p2_t, "fwd", "fwd")
    sub0 = _knn(p1, pts_t, "mid", "fwd")
    sub1 = _knn(p2, p1_t, "fwd", "mid")
    up0 = _knn(pts, p1_t, "fwd", "mid")
    up1 = _knn(p1, p2_t, "mid", "fwd")
    return (pts, p1, p2, nb0, nb1, nb2, sub0, sub1, up0, up1)


# speedup vs baseline: 6.8894x; 5.8198x over previous
import functools

import jax
import jax.numpy as jnp
from jax.experimental import pallas as pl
from jax.experimental.pallas import tpu as pltpu

K = 16


def _fps_kernel(pts_t_ref, out_ref):
    B, _, N = pts_t_ref.shape
    M = out_ref.shape[1]
    X = pts_t_ref[:, 0, :]
    Y = pts_t_ref[:, 1, :]
    Z = pts_t_ref[:, 2, :]
    iota = jax.lax.broadcasted_iota(jnp.int32, (B, N), 1)

    def row(cx, cy, cz):
        return jnp.concatenate(
            [cx.reshape(B, 1, 1), cy.reshape(B, 1, 1), cz.reshape(B, 1, 1)],
            axis=2)

    lx, ly, lz = X[:, 0:1], Y[:, 0:1], Z[:, 0:1]
    out_ref[:, 0:1, :] = row(lx, ly, lz)
    mind0 = jnp.full((B, N), 1e10, dtype=jnp.float32)

    def step(j, carry):
        mind, lx, ly, lz = carry
        dx = X - lx
        dy = Y - ly
        dz = Z - lz
        d = (dx * dx + dz * dz) + dy * dy
        mind = jnp.minimum(mind, d)
        m = jnp.max(mind, axis=1, keepdims=True)
        am = jnp.min(jnp.where(mind == m, iota, N), axis=1, keepdims=True)
        oh = iota == am
        nlx = jnp.sum(jnp.where(oh, X, 0.0), axis=1, keepdims=True)
        nly = jnp.sum(jnp.where(oh, Y, 0.0), axis=1, keepdims=True)
        nlz = jnp.sum(jnp.where(oh, Z, 0.0), axis=1, keepdims=True)
        out_ref[:, pl.ds(j, 1), :] = row(nlx, nly, nlz)
        return (mind, nlx, nly, nlz)

    jax.lax.fori_loop(1, M, step, (mind0, lx, ly, lz), unroll=False)


def _fps(pts_t, m):
    B = pts_t.shape[0]
    return pl.pallas_call(
        _fps_kernel,
        out_shape=jax.ShapeDtypeStruct((B, m, 3), jnp.float32),
    )(pts_t)


def _sqnorm(x, y, z, order):
    if order == "fwd":
        return (x * x + y * y) + z * z
    return (x * x + z * z) + y * y


def _knn_kernel(q_ref, st_ref, o_ref, *, q_order, s_order):
    R = q_ref.shape[1]
    S = st_ref.shape[2]
    qx, qy, qz = q_ref[0, :, 0:1], q_ref[0, :, 1:2], q_ref[0, :, 2:3]
    sx, sy, sz = st_ref[0, 0:1, :], st_ref[0, 1:2, :], st_ref[0, 2:3, :]
    q2 = _sqnorm(qx, qy, qz, q_order)
    s2 = _sqnorm(sx, sy, sz, s_order)
    qs = jax.lax.dot_general(
        q_ref[0], st_ref[0], (((1,), (0,)), ((), ())),
        precision=jax.lax.Precision.DEFAULT,
        preferred_element_type=jnp.float32)
    d = (q2 + s2) - 2.0 * qs
    iota = jax.lax.broadcasted_iota(jnp.int32, (R, S), 1)
    big = jnp.float32(jnp.inf)
    for j in range(K):
        m = jnp.min(d, axis=1, keepdims=True)
        am = jnp.min(jnp.where(d == m, iota, S), axis=1, keepdims=True)
        o_ref[0, :, j:j + 1] = am
        if j + 1 < K:
            d = jnp.where(iota == am, big, d)


def _knn(q_rows, st, q_order, s_order):
    B, Q, _ = q_rows.shape
    S = st.shape[2]
    R = min(Q, 64)
    return pl.pallas_call(
        functools.partial(_knn_kernel, q_order=q_order, s_order=s_order),
        grid=(B, Q // R),
        in_specs=[
            pl.BlockSpec((1, R, 3), lambda b, i: (b, i, 0)),
            pl.BlockSpec((1, 3, S), lambda b, i: (b, 0, 0)),
        ],
        out_specs=pl.BlockSpec((1, R, K), lambda b, i: (b, i, 0)),
        out_shape=jax.ShapeDtypeStruct((B, Q, K), jnp.int32),
    )(q_rows, st)


def kernel(points):
    pts = points[..., :3]
    B, n0, _ = pts.shape
    n1, n2 = n0 // 4, n0 // 16
    pts_t = jnp.swapaxes(pts, 1, 2)
    p1 = _fps(pts_t, n1)
    p1_t = jnp.swapaxes(p1, 1, 2)
    p2 = _fps(p1_t, n2)
    p2_t = jnp.swapaxes(p2, 1, 2)
    nb0 = _knn(pts, pts_t, "fwd", "fwd")
    nb1 = _knn(p1, p1_t, "mid", "mid")
    nb2 = _knn(p2, p2_t, "fwd", "fwd")
    sub0 = _knn(p1, pts_t, "mid", "fwd")
    sub1 = _knn(p2, p1_t, "fwd", "mid")
    up0 = _knn(pts, p1_t, "fwd", "mid")
    up1 = _knn(p1, p2_t, "mid", "fwd")
    return (pts, p1, p2, nb0, nb1, nb2, sub0, sub1, up0, up1)


# speedup vs baseline: 9.4321x; 1.3691x over previous
import functools

import jax
import jax.numpy as jnp
from jax.experimental import pallas as pl
from jax.experimental.pallas import tpu as pltpu

K = 16


def _fps_kernel(pts_t_ref, out_ref):
    B, _, N = pts_t_ref.shape
    M = out_ref.shape[1]
    X = pts_t_ref[:, 0, :]
    Y = pts_t_ref[:, 1, :]
    Z = pts_t_ref[:, 2, :]
    iota = jax.lax.broadcasted_iota(jnp.int32, (B, N), 1)

    def row(cx, cy, cz):
        return jnp.concatenate(
            [cx.reshape(B, 1, 1), cy.reshape(B, 1, 1), cz.reshape(B, 1, 1)],
            axis=2)

    lx, ly, lz = X[:, 0:1], Y[:, 0:1], Z[:, 0:1]
    out_ref[:, 0:1, :] = row(lx, ly, lz)
    mind0 = jnp.full((B, N), 1e10, dtype=jnp.float32)

    def step(j, carry):
        mind, lx, ly, lz = carry
        dx = X - lx
        dy = Y - ly
        dz = Z - lz
        d = (dx * dx + dz * dz) + dy * dy
        mind = jnp.minimum(mind, d)
        m = jnp.max(mind, axis=1, keepdims=True)
        am = jnp.min(jnp.where(mind == m, iota, N), axis=1, keepdims=True)
        oh = iota == am
        nlx = jnp.sum(jnp.where(oh, X, 0.0), axis=1, keepdims=True)
        nly = jnp.sum(jnp.where(oh, Y, 0.0), axis=1, keepdims=True)
        nlz = jnp.sum(jnp.where(oh, Z, 0.0), axis=1, keepdims=True)
        out_ref[:, pl.ds(j, 1), :] = row(nlx, nly, nlz)
        return (mind, nlx, nly, nlz)

    jax.lax.fori_loop(1, M, step, (mind0, lx, ly, lz), unroll=False)


def _fps(pts_t, m):
    B = pts_t.shape[0]
    return pl.pallas_call(
        _fps_kernel,
        out_shape=jax.ShapeDtypeStruct((B, m, 3), jnp.float32),
    )(pts_t)


def _sqnorm(x, y, z, order):
    if order == "fwd":
        return (x * x + y * y) + z * z
    return (x * x + z * z) + y * y


def _knn_kernel(q_ref, st_ref, o_ref, *, q_order, s_order):
    R = q_ref.shape[1]
    S = st_ref.shape[2]
    qx, qy, qz = q_ref[0, :, 0:1], q_ref[0, :, 1:2], q_ref[0, :, 2:3]
    sx, sy, sz = st_ref[0, 0:1, :], st_ref[0, 1:2, :], st_ref[0, 2:3, :]
    q2 = _sqnorm(qx, qy, qz, q_order)
    s2 = _sqnorm(sx, sy, sz, s_order)
    qs = jax.lax.dot_general(
        q_ref[0], st_ref[0], (((1,), (0,)), ((), ())),
        precision=jax.lax.Precision.DEFAULT,
        preferred_element_type=jnp.float32)
    d = (q2 + s2) - 2.0 * qs
    iota = jax.lax.broadcasted_iota(jnp.int32, (R, S), 1)
    big = jnp.float32(jnp.inf)
    for j in range(K):
        m = jnp.min(d, axis=1, keepdims=True)
        am = jnp.min(jnp.where(d == m, iota, S), axis=1, keepdims=True)
        o_ref[0, :, j:j + 1] = am
        if j + 1 < K:
            d = jnp.where(iota == am, big, d)


def _knn(q_rows, st, q_order, s_order):
    B, Q, _ = q_rows.shape
    S = st.shape[2]
    R = min(Q, 128)
    return pl.pallas_call(
        functools.partial(_knn_kernel, q_order=q_order, s_order=s_order),
        grid=(B, Q // R),
        in_specs=[
            pl.BlockSpec((1, R, 3), lambda b, i: (b, i, 0)),
            pl.BlockSpec((1, 3, S), lambda b, i: (b, 0, 0)),
        ],
        out_specs=pl.BlockSpec((1, R, K), lambda b, i: (b, i, 0)),
        out_shape=jax.ShapeDtypeStruct((B, Q, K), jnp.int32),
    )(q_rows, st)


def kernel(points):
    pts = points[..., :3]
    B, n0, _ = pts.shape
    n1, n2 = n0 // 4, n0 // 16
    pts_t = jnp.swapaxes(pts, 1, 2)
    p1 = _fps(pts_t, n1)
    p1_t = jnp.swapaxes(p1, 1, 2)
    p2 = _fps(p1_t, n2)
    p2_t = jnp.swapaxes(p2, 1, 2)
    nb0 = _knn(pts, pts_t, "fwd", "fwd")
    nb1 = _knn(p1, p1_t, "mid", "mid")
    nb2 = _knn(p2, p2_t, "fwd", "fwd")
    sub0 = _knn(p1, pts_t, "mid", "fwd")
    sub1 = _knn(p2, p1_t, "fwd", "mid")
    up0 = _knn(pts, p1_t, "fwd", "mid")
    up1 = _knn(p1, p2_t, "mid", "fwd")
    return (pts, p1, p2, nb0, nb1, nb2, sub0, sub1, up0, up1)


# speedup vs baseline: 11.2517x; 1.1929x over previous
import functools

import jax
import jax.numpy as jnp
from jax.experimental import pallas as pl
from jax.experimental.pallas import tpu as pltpu

K = 16


def _fps_kernel(pts_t_ref, out_ref):
    B, _, N = pts_t_ref.shape
    M = out_ref.shape[1]
    X = pts_t_ref[:, 0, :]
    Y = pts_t_ref[:, 1, :]
    Z = pts_t_ref[:, 2, :]
    iota = jax.lax.broadcasted_iota(jnp.int32, (B, N), 1)

    def row(cx, cy, cz):
        return jnp.concatenate(
            [cx.reshape(B, 1, 1), cy.reshape(B, 1, 1), cz.reshape(B, 1, 1)],
            axis=2)

    lx, ly, lz = X[:, 0:1], Y[:, 0:1], Z[:, 0:1]
    out_ref[:, 0:1, :] = row(lx, ly, lz)
    mind0 = jnp.full((B, N), 1e10, dtype=jnp.float32)

    def step(j, carry):
        mind, lx, ly, lz = carry
        dx = X - lx
        dy = Y - ly
        dz = Z - lz
        d = (dx * dx + dz * dz) + dy * dy
        mind = jnp.minimum(mind, d)
        m = jnp.max(mind, axis=1, keepdims=True)
        am = jnp.min(jnp.where(mind == m, iota, N), axis=1, keepdims=True)
        oh = iota == am
        nlx = jnp.sum(jnp.where(oh, X, 0.0), axis=1, keepdims=True)
        nly = jnp.sum(jnp.where(oh, Y, 0.0), axis=1, keepdims=True)
        nlz = jnp.sum(jnp.where(oh, Z, 0.0), axis=1, keepdims=True)
        out_ref[:, pl.ds(j, 1), :] = row(nlx, nly, nlz)
        return (mind, nlx, nly, nlz)

    jax.lax.fori_loop(1, M, step, (mind0, lx, ly, lz), unroll=False)


def _fps(pts_t, m):
    B = pts_t.shape[0]
    return pl.pallas_call(
        _fps_kernel,
        out_shape=jax.ShapeDtypeStruct((B, m, 3), jnp.float32),
    )(pts_t)


def _sqnorm(x, y, z, order):
    if order == "fwd":
        return (x * x + y * y) + z * z
    return (x * x + z * z) + y * y


def _knn_kernel(q_ref, st_ref, o_ref, *, q_order, s_order):
    R = q_ref.shape[1]
    S = st_ref.shape[2]
    qx, qy, qz = q_ref[0, :, 0:1], q_ref[0, :, 1:2], q_ref[0, :, 2:3]
    sx, sy, sz = st_ref[0, 0:1, :], st_ref[0, 1:2, :], st_ref[0, 2:3, :]
    q2 = _sqnorm(qx, qy, qz, q_order)
    s2 = _sqnorm(sx, sy, sz, s_order)
    qs = jax.lax.dot_general(
        q_ref[0], st_ref[0], (((1,), (0,)), ((), ())),
        precision=jax.lax.Precision.DEFAULT,
        preferred_element_type=jnp.float32)
    d = (q2 + s2) - 2.0 * qs
    iota = jax.lax.broadcasted_iota(jnp.int32, (R, S), 1)
    big = jnp.float32(jnp.inf)
    for j in range(K):
        m = jnp.min(d, axis=1, keepdims=True)
        am = jnp.min(jnp.where(d == m, iota, S), axis=1, keepdims=True)
        o_ref[0, :, j:j + 1] = am
        if j + 1 < K:
            d = jnp.where(iota == am, big, d)


def _knn(q_rows, st, q_order, s_order):
    B, Q, _ = q_rows.shape
    S = st.shape[2]
    R = min(Q, 256)
    return pl.pallas_call(
        functools.partial(_knn_kernel, q_order=q_order, s_order=s_order),
        grid=(B, Q // R),
        in_specs=[
            pl.BlockSpec((1, R, 3), lambda b, i: (b, i, 0)),
            pl.BlockSpec((1, 3, S), lambda b, i: (b, 0, 0)),
        ],
        out_specs=pl.BlockSpec((1, R, K), lambda b, i: (b, i, 0)),
        out_shape=jax.ShapeDtypeStruct((B, Q, K), jnp.int32),
    )(q_rows, st)


def kernel(points):
    pts = points[..., :3]
    B, n0, _ = pts.shape
    n1, n2 = n0 // 4, n0 // 16
    pts_t = jnp.swapaxes(pts, 1, 2)
    p1 = _fps(pts_t, n1)
    p1_t = jnp.swapaxes(p1, 1, 2)
    p2 = _fps(p1_t, n2)
    p2_t = jnp.swapaxes(p2, 1, 2)
    nb0 = _knn(pts, pts_t, "fwd", "fwd")
    nb1 = _knn(p1, p1_t, "mid", "mid")
    nb2 = _knn(p2, p2_t, "fwd", "fwd")
    sub0 = _knn(p1, pts_t, "mid", "fwd")
    sub1 = _knn(p2, p1_t, "fwd", "mid")
    up0 = _knn(pts, p1_t, "fwd", "mid")
    up1 = _knn(p1, p2_t, "mid", "fwd")
    return (pts, p1, p2, nb0, nb1, nb2, sub0, sub1, up0, up1)


# speedup vs baseline: 11.8996x; 1.0576x over previous
import functools

import jax
import jax.numpy as jnp
from jax.experimental import pallas as pl
from jax.experimental.pallas import tpu as pltpu

K = 16


def _fps_kernel(pts_t_ref, out_ref):
    B, _, N = pts_t_ref.shape
    M = out_ref.shape[1]
    X = pts_t_ref[:, 0, :]
    Y = pts_t_ref[:, 1, :]
    Z = pts_t_ref[:, 2, :]
    iota = jax.lax.broadcasted_iota(jnp.int32, (B, N), 1)

    def row(cx, cy, cz):
        return jnp.concatenate(
            [cx.reshape(B, 1, 1), cy.reshape(B, 1, 1), cz.reshape(B, 1, 1)],
            axis=2)

    lx, ly, lz = X[:, 0:1], Y[:, 0:1], Z[:, 0:1]
    out_ref[:, 0:1, :] = row(lx, ly, lz)
    mind0 = jnp.full((B, N), 1e10, dtype=jnp.float32)

    def step(j, carry):
        mind, lx, ly, lz = carry
        dx = X - lx
        dy = Y - ly
        dz = Z - lz
        d = (dx * dx + dz * dz) + dy * dy
        mind = jnp.minimum(mind, d)
        m = jnp.max(mind, axis=1, keepdims=True)
        am = jnp.min(jnp.where(mind == m, iota, N), axis=1, keepdims=True)
        oh = iota == am
        nlx = jnp.sum(jnp.where(oh, X, 0.0), axis=1, keepdims=True)
        nly = jnp.sum(jnp.where(oh, Y, 0.0), axis=1, keepdims=True)
        nlz = jnp.sum(jnp.where(oh, Z, 0.0), axis=1, keepdims=True)
        out_ref[:, pl.ds(j, 1), :] = row(nlx, nly, nlz)
        return (mind, nlx, nly, nlz)

    jax.lax.fori_loop(1, M, step, (mind0, lx, ly, lz), unroll=False)


def _fps(pts_t, m):
    B = pts_t.shape[0]
    return pl.pallas_call(
        _fps_kernel,
        out_shape=jax.ShapeDtypeStruct((B, m, 3), jnp.float32),
    )(pts_t)


def _sqnorm(x, y, z, order):
    if order == "fwd":
        return (x * x + y * y) + z * z
    return (x * x + z * z) + y * y


def _knn_kernel(q_ref, st_ref, o_ref, *, q_order, s_order):
    R = q_ref.shape[1]
    S = st_ref.shape[2]
    qx, qy, qz = q_ref[0, :, 0:1], q_ref[0, :, 1:2], q_ref[0, :, 2:3]
    sx, sy, sz = st_ref[0, 0:1, :], st_ref[0, 1:2, :], st_ref[0, 2:3, :]
    q2 = _sqnorm(qx, qy, qz, q_order)
    s2 = _sqnorm(sx, sy, sz, s_order)
    qs = jax.lax.dot_general(
        q_ref[0], st_ref[0], (((1,), (0,)), ((), ())),
        precision=jax.lax.Precision.DEFAULT,
        preferred_element_type=jnp.float32)
    d = (q2 + s2) - 2.0 * qs
    iota = jax.lax.broadcasted_iota(jnp.int32, (R, S), 1)
    big = jnp.float32(jnp.inf)
    for j in range(K):
        m = jnp.min(d, axis=1, keepdims=True)
        am = jnp.min(jnp.where(d == m, iota, S), axis=1, keepdims=True)
        o_ref[0, :, j:j + 1] = am
        if j + 1 < K:
            d = jnp.where(iota == am, big, d)


def _knn(q_rows, st, q_order, s_order):
    B, Q, _ = q_rows.shape
    S = st.shape[2]
    R = min(Q, 512)
    return pl.pallas_call(
        functools.partial(_knn_kernel, q_order=q_order, s_order=s_order),
        grid=(B, Q // R),
        in_specs=[
            pl.BlockSpec((1, R, 3), lambda b, i: (b, i, 0)),
            pl.BlockSpec((1, 3, S), lambda b, i: (b, 0, 0)),
        ],
        out_specs=pl.BlockSpec((1, R, K), lambda b, i: (b, i, 0)),
        out_shape=jax.ShapeDtypeStruct((B, Q, K), jnp.int32),
    )(q_rows, st)


def kernel(points):
    pts = points[..., :3]
    B, n0, _ = pts.shape
    n1, n2 = n0 // 4, n0 // 16
    pts_t = jnp.swapaxes(pts, 1, 2)
    p1 = _fps(pts_t, n1)
    p1_t = jnp.swapaxes(p1, 1, 2)
    p2 = _fps(p1_t, n2)
    p2_t = jnp.swapaxes(p2, 1, 2)
    nb0 = _knn(pts, pts_t, "fwd", "fwd")
    nb1 = _knn(p1, p1_t, "mid", "mid")
    nb2 = _knn(p2, p2_t, "fwd", "fwd")
    sub0 = _knn(p1, pts_t, "mid", "fwd")
    sub1 = _knn(p2, p1_t, "fwd", "mid")
    up0 = _knn(pts, p1_t, "fwd", "mid")
    up1 = _knn(p1, p2_t, "mid", "fwd")
    return (pts, p1, p2, nb0, nb1, nb2, sub0, sub1, up0, up1)


# speedup vs baseline: 11.8996x; 1.0000x over previous
import functools

import jax
import jax.numpy as jnp
from jax.experimental import pallas as pl
from jax.experimental.pallas import tpu as pltpu

K = 16


def _fps_kernel(pts_t_ref, out_ref):
    B, _, N = pts_t_ref.shape
    M = out_ref.shape[1]
    X = pts_t_ref[:, 0, :]
    Y = pts_t_ref[:, 1, :]
    Z = pts_t_ref[:, 2, :]
    iota = jax.lax.broadcasted_iota(jnp.int32, (B, N), 1)

    def row(cx, cy, cz):
        return jnp.concatenate(
            [cx.reshape(B, 1, 1), cy.reshape(B, 1, 1), cz.reshape(B, 1, 1)],
            axis=2)

    lx, ly, lz = X[:, 0:1], Y[:, 0:1], Z[:, 0:1]
    out_ref[:, 0:1, :] = row(lx, ly, lz)
    mind0 = jnp.full((B, N), 1e10, dtype=jnp.float32)

    def step(j, carry):
        mind, lx, ly, lz = carry
        dx = X - lx
        dy = Y - ly
        dz = Z - lz
        d = (dx * dx + dz * dz) + dy * dy
        mind = jnp.minimum(mind, d)
        m = jnp.max(mind, axis=1, keepdims=True)
        am = jnp.min(jnp.where(mind == m, iota, N), axis=1, keepdims=True)
        oh = iota == am
        nlx = jnp.sum(jnp.where(oh, X, 0.0), axis=1, keepdims=True)
        nly = jnp.sum(jnp.where(oh, Y, 0.0), axis=1, keepdims=True)
        nlz = jnp.sum(jnp.where(oh, Z, 0.0), axis=1, keepdims=True)
        out_ref[:, pl.ds(j, 1), :] = row(nlx, nly, nlz)
        return (mind, nlx, nly, nlz)

    jax.lax.fori_loop(1, M, step, (mind0, lx, ly, lz), unroll=False)


def _fps(pts_t, m):
    B = pts_t.shape[0]
    return pl.pallas_call(
        _fps_kernel,
        out_shape=jax.ShapeDtypeStruct((B, m, 3), jnp.float32),
    )(pts_t)


def _sqnorm(x, y, z, order):
    if order == "fwd":
        return (x * x + y * y) + z * z
    return (x * x + z * z) + y * y


def _knn_kernel(q_ref, st_ref, o_ref, *, q_order, s_order):
    R = q_ref.shape[1]
    S = st_ref.shape[2]
    qx, qy, qz = q_ref[0, :, 0:1], q_ref[0, :, 1:2], q_ref[0, :, 2:3]
    sx, sy, sz = st_ref[0, 0:1, :], st_ref[0, 1:2, :], st_ref[0, 2:3, :]
    q2 = _sqnorm(qx, qy, qz, q_order)
    s2 = _sqnorm(sx, sy, sz, s_order)
    qs = jax.lax.dot_general(
        q_ref[0], st_ref[0], (((1,), (0,)), ((), ())),
        precision=jax.lax.Precision.DEFAULT,
        preferred_element_type=jnp.float32)
    d = (q2 + s2) - 2.0 * qs
    iota = jax.lax.broadcasted_iota(jnp.int32, (R, S), 1)
    big = jnp.float32(jnp.inf)
    am = None
    for j in range(K):
        if j > 0:
            d = jnp.where(iota == am, big, d)
        m = jnp.min(d, axis=1, keepdims=True)
        am = jnp.min(jnp.where(d == m, iota, S), axis=1, keepdims=True)
        o_ref[0, :, j:j + 1] = am


def _knn(q_rows, st, q_order, s_order):
    B, Q, _ = q_rows.shape
    S = st.shape[2]
    R = min(Q, 512)
    return pl.pallas_call(
        functools.partial(_knn_kernel, q_order=q_order, s_order=s_order),
        grid=(B, Q // R),
        in_specs=[
            pl.BlockSpec((1, R, 3), lambda b, i: (b, i, 0)),
            pl.BlockSpec((1, 3, S), lambda b, i: (b, 0, 0)),
        ],
        out_specs=pl.BlockSpec((1, R, K), lambda b, i: (b, i, 0)),
        out_shape=jax.ShapeDtypeStruct((B, Q, K), jnp.int32),
    )(q_rows, st)


def kernel(points):
    pts = points[..., :3]
    B, n0, _ = pts.shape
    n1, n2 = n0 // 4, n0 // 16
    pts_t = jnp.swapaxes(pts, 1, 2)
    p1 = _fps(pts_t, n1)
    p1_t = jnp.swapaxes(p1, 1, 2)
    p2 = _fps(p1_t, n2)
    p2_t = jnp.swapaxes(p2, 1, 2)
    nb0 = _knn(pts, pts_t, "fwd", "fwd")
    nb1 = _knn(p1, p1_t, "mid", "mid")
    nb2 = _knn(p2, p2_t, "fwd", "fwd")
    sub0 = _knn(p1, pts_t, "mid", "fwd")
    sub1 = _knn(p2, p1_t, "fwd", "mid")
    up0 = _knn(pts, p1_t, "fwd", "mid")
    up1 = _knn(p1, p2_t, "mid", "fwd")
    return (pts, p1, p2, nb0, nb1, nb2, sub0, sub1, up0, up1)


# speedup vs baseline: 11.9007x; 1.0001x over previous
import functools

import jax
import jax.numpy as jnp
from jax.experimental import pallas as pl

K = 16


def _fps_kernel(pts_t_ref, out_ref):
    B, _, N = pts_t_ref.shape
    M = out_ref.shape[1]
    X = pts_t_ref[:, 0, :]
    Y = pts_t_ref[:, 1, :]
    Z = pts_t_ref[:, 2, :]
    iota = jax.lax.broadcasted_iota(jnp.int32, (B, N), 1)

    def row(cx, cy, cz):
        return jnp.concatenate(
            [cx.reshape(B, 1, 1), cy.reshape(B, 1, 1), cz.reshape(B, 1, 1)],
            axis=2)

    lx, ly, lz = X[:, 0:1], Y[:, 0:1], Z[:, 0:1]
    out_ref[:, 0:1, :] = row(lx, ly, lz)
    mind0 = jnp.full((B, N), 1e10, dtype=jnp.float32)

    def step(j, carry):
        mind, lx, ly, lz = carry
        dx = X - lx
        dy = Y - ly
        dz = Z - lz
        d = (dx * dx + dz * dz) + dy * dy
        mind = jnp.minimum(mind, d)
        m = jnp.max(mind, axis=1, keepdims=True)
        am = jnp.min(jnp.where(mind == m, iota, N), axis=1, keepdims=True)
        oh = iota == am
        nlx = jnp.sum(jnp.where(oh, X, 0.0), axis=1, keepdims=True)
        nly = jnp.sum(jnp.where(oh, Y, 0.0), axis=1, keepdims=True)
        nlz = jnp.sum(jnp.where(oh, Z, 0.0), axis=1, keepdims=True)
        out_ref[:, pl.ds(j, 1), :] = row(nlx, nly, nlz)
        return (mind, nlx, nly, nlz)

    jax.lax.fori_loop(1, M, step, (mind0, lx, ly, lz), unroll=False)


def _fps(pts_t, m):
    B = pts_t.shape[0]
    return pl.pallas_call(
        _fps_kernel,
        out_shape=jax.ShapeDtypeStruct((B, m, 3), jnp.float32),
    )(pts_t)


def _sqnorm(x, y, z, order):
    if order == "fwd":
        return (x * x + y * y) + z * z
    return (x * x + z * z) + y * y


def _knn_kernel(q_ref, st_ref, o_ref, *, q_order, s_order):
    R = q_ref.shape[1]
    S = st_ref.shape[2]
    qx, qy, qz = q_ref[0, :, 0:1], q_ref[0, :, 1:2], q_ref[0, :, 2:3]
    sx, sy, sz = st_ref[0, 0:1, :], st_ref[0, 1:2, :], st_ref[0, 2:3, :]
    q2 = _sqnorm(qx, qy, qz, q_order)
    s2 = _sqnorm(sx, sy, sz, s_order)
    qs = jax.lax.dot_general(
        q_ref[0], st_ref[0], (((1,), (0,)), ((), ())),
        precision=jax.lax.Precision.DEFAULT,
        preferred_element_type=jnp.float32)
    d = (q2 + s2) - 2.0 * qs
    iota = jax.lax.broadcasted_iota(jnp.int32, (R, S), 1)
    big = jnp.float32(jnp.inf)
    am = None
    for j in range(K):
        if j > 0:
            d = jnp.where(iota == am, big, d)
        m = jnp.min(d, axis=1, keepdims=True)
        am = jnp.min(jnp.where(d == m, iota, S), axis=1, keepdims=True)
        o_ref[0, :, j:j + 1] = am


def _knn(q_rows, st, q_order, s_order):
    B, Q, _ = q_rows.shape
    S = st.shape[2]
    R = min(Q, 512)
    return pl.pallas_call(
        functools.partial(_knn_kernel, q_order=q_order, s_order=s_order),
        grid=(B, Q // R),
        in_specs=[
            pl.BlockSpec((1, R, 3), lambda b, i: (b, i, 0)),
            pl.BlockSpec((1, 3, S), lambda b, i: (b, 0, 0)),
        ],
        out_specs=pl.BlockSpec((1, R, K), lambda b, i: (b, i, 0)),
        out_shape=jax.ShapeDtypeStruct((B, Q, K), jnp.int32),
    )(q_rows, st)


def kernel(points):
    pts = points[..., :3]
    B, n0, _ = pts.shape
    n1, n2 = n0 // 4, n0 // 16
    pts_t = jnp.swapaxes(pts, 1, 2)
    p1 = _fps(pts_t, n1)
    p1_t = jnp.swapaxes(p1, 1, 2)
    p2 = _fps(p1_t, n2)
    p2_t = jnp.swapaxes(p2, 1, 2)
    nb0 = _knn(pts, pts_t, "fwd", "fwd")
    nb1 = _knn(p1, p1_t, "mid", "mid")
    nb2 = _knn(p2, p2_t, "fwd", "fwd")
    sub0 = _knn(p1, pts_t, "mid", "fwd")
    sub1 = _knn(p2, p1_t, "fwd", "mid")
    up0 = _knn(pts, p1_t, "fwd", "mid")
    up1 = _knn(p1, p2_t, "mid", "fwd")
    return (pts, p1, p2, nb0, nb1, nb2, sub0, sub1, up0, up1)
